# speedup vs baseline: 20.3662x; 20.3662x over previous
import functools

import jax
import jax.numpy as jnp
from jax import lax
from jax.experimental import pallas as pl
from jax.experimental.pallas import tpu as pltpu

NT = 512


def _k0_body(p2_ref, wb_ref, out_ref):
    p2 = p2_ref[0]
    out_ref[0] = lax.dot_general(
        p2, wb_ref[...], (((0,), (1,)), ((), ())),
        preferred_element_type=jnp.float32)


def _k1_body(x1_ref, x2_ref, p1_ref, p2w_ref, wa_ref, fb_ref,
             y1_ref, stats_ref, *, S):
    b = pl.program_id(0)
    i = pl.program_id(1)
    x1 = x1_ref[0]
    x2 = x2_ref[0]
    dist = -2.0 * lax.dot_general(
        x1, x2, (((1,), (1,)), ((), ())), preferred_element_type=jnp.float32)
    dist = dist + jnp.sum(x1 * x1, axis=1, keepdims=True)
    dist = dist + jnp.sum(x2 * x2, axis=1)[None, :]

    iota = lax.broadcasted_iota(jnp.int32, (NT, S), 1)
    cur = dist
    recips = []
    sels = []
    for _ in range(3):
        mn = jnp.min(cur, axis=1, keepdims=True)
        idxk = jnp.min(jnp.where(cur == mn, iota, S), axis=1,
                       keepdims=True)
        sel = iota == idxk
        recips.append(1.0 / (mn + 1e-8))
        sels.append(sel)
        cur = jnp.where(sel, jnp.inf, cur)
    norm = recips[0] + recips[1] + recips[2]
    wsp = jnp.zeros((NT, S), jnp.float32)
    for r, sel in zip(recips, sels):
        wsp = wsp + jnp.where(sel, r / norm, 0.0)

    y = lax.dot_general(wsp, p2w_ref[0], (((1,), (0,)), ((), ())),
                        preferred_element_type=jnp.float32)
    y = y + lax.dot_general(p1_ref[0], wa_ref[...], (((0,), (1,)), ((), ())),
                            preferred_element_type=jnp.float32)
    y = y + fb_ref[...]
    y1_ref[...] = y

    @pl.when(jnp.logical_and(b == 0, i == 0))
    def _():
        stats_ref[...] = jnp.zeros_like(stats_ref)

    stats_ref[0:1, :] = stats_ref[0:1, :] + jnp.sum(y, axis=0, keepdims=True)
    stats_ref[1:2, :] = stats_ref[1:2, :] + jnp.sum(y * y, axis=0,
                                                    keepdims=True)


def _bn_coeffs(stats, g, beta, count):
    mean = stats[0:1, :] / count
    var = stats[1:2, :] / count - mean * mean
    a = g / jnp.sqrt(var + 1e-5)
    c = beta - mean * a
    return a, c


def _k2_body(y1_ref, stats_ref, g_ref, beta_ref, w_ref, bb_ref,
             z_ref, y2_ref, stats2_ref, *, count):
    a, c = _bn_coeffs(stats_ref[...], g_ref[...], beta_ref[...], count)
    z = jnp.maximum(y1_ref[...] * a + c, 0.0)
    z_ref[...] = z
    y2 = lax.dot_general(z, w_ref[...], (((1,), (1,)), ((), ())),
                         preferred_element_type=jnp.float32) + bb_ref[...]
    y2_ref[...] = y2

    @pl.when(pl.program_id(0) == 0)
    def _():
        stats2_ref[...] = jnp.zeros_like(stats2_ref)

    stats2_ref[0:1, :] = stats2_ref[0:1, :] + jnp.sum(y2, axis=0,
                                                      keepdims=True)
    stats2_ref[1:2, :] = stats2_ref[1:2, :] + jnp.sum(y2 * y2, axis=0,
                                                      keepdims=True)


def _k3_body(y2_ref, stats_ref, g_ref, beta_ref, w_ref, bb_ref,
             y3_ref, stats3_ref, *, count):
    a, c = _bn_coeffs(stats_ref[...], g_ref[...], beta_ref[...], count)
    z2 = jnp.maximum(y2_ref[...] * a + c, 0.0)
    y3 = lax.dot_general(z2, w_ref[...], (((1,), (1,)), ((), ())),
                         preferred_element_type=jnp.float32) + bb_ref[...]
    y3_ref[...] = y3

    @pl.when(pl.program_id(0) == 0)
    def _():
        stats3_ref[...] = jnp.zeros_like(stats3_ref)

    stats3_ref[0:1, :] = stats3_ref[0:1, :] + jnp.sum(y3, axis=0,
                                                      keepdims=True)
    stats3_ref[1:2, :] = stats3_ref[1:2, :] + jnp.sum(y3 * y3, axis=0,
                                                      keepdims=True)


def _k4_body(y3_ref, z_ref, stats_ref, g_ref, beta_ref, out_ref, *, count):
    a, c = _bn_coeffs(stats_ref[...], g_ref[...], beta_ref[...], count)
    o = jnp.maximum(y3_ref[...] * a + c + z_ref[...], 0.0)
    out_ref[0] = o.T


def kernel(xyz1, xyz2, points1, points2, fuse_W, fuse_b, fuse_g, fuse_beta,
           w1, b1, g1, be1, w2, b2, g2, be2):
    B, _, N = xyz1.shape
    S = xyz2.shape[2]
    D1 = points1.shape[1]
    O = fuse_W.shape[0]
    NBLK = N // NT
    ROWS = B * N
    RBLK = ROWS // NT
    count = float(ROWS)

    x1t = jnp.transpose(xyz1, (0, 2, 1))
    x2t = jnp.transpose(xyz2, (0, 2, 1))
    wa = fuse_W[:, :D1]
    wb = fuse_W[:, D1:]
    fb = fuse_b[None, :]
    f32 = jnp.float32

    p2w = pl.pallas_call(
        _k0_body,
        grid=(B,),
        in_specs=[
            pl.BlockSpec((1, points2.shape[1], S), lambda b: (b, 0, 0)),
            pl.BlockSpec((O, points2.shape[1]), lambda b: (0, 0)),
        ],
        out_specs=pl.BlockSpec((1, S, O), lambda b: (b, 0, 0)),
        out_shape=jax.ShapeDtypeStruct((B, S, O), f32),
    )(points2, wb)

    y1, stats1 = pl.pallas_call(
        functools.partial(_k1_body, S=S),
        grid=(B, NBLK),
        in_specs=[
            pl.BlockSpec((1, NT, 3), lambda b, i: (b, i, 0)),
            pl.BlockSpec((1, S, 3), lambda b, i: (b, 0, 0)),
            pl.BlockSpec((1, D1, NT), lambda b, i: (b, 0, i)),
            pl.BlockSpec((1, S, O), lambda b, i: (b, 0, 0)),
            pl.BlockSpec((O, D1), lambda b, i: (0, 0)),
            pl.BlockSpec((1, O), lambda b, i: (0, 0)),
        ],
        out_specs=[
            pl.BlockSpec((NT, O), lambda b, i: (b * NBLK + i, 0)),
            pl.BlockSpec((8, O), lambda b, i: (0, 0)),
        ],
        out_shape=[
            jax.ShapeDtypeStruct((ROWS, O), f32),
            jax.ShapeDtypeStruct((8, O), f32),
        ],
    )(x1t, x2t, points1, p2w, wa, fb)

    row_specs = [
        pl.BlockSpec((NT, O), lambda i: (i, 0)),
        pl.BlockSpec((8, O), lambda i: (0, 0)),
        pl.BlockSpec((1, O), lambda i: (0, 0)),
        pl.BlockSpec((1, O), lambda i: (0, 0)),
        pl.BlockSpec((O, O), lambda i: (0, 0)),
        pl.BlockSpec((1, O), lambda i: (0, 0)),
    ]
    z, y2, stats2 = pl.pallas_call(
        functools.partial(_k2_body, count=count),
        grid=(RBLK,),
        in_specs=row_specs,
        out_specs=[
            pl.BlockSpec((NT, O), lambda i: (i, 0)),
            pl.BlockSpec((NT, O), lambda i: (i, 0)),
            pl.BlockSpec((8, O), lambda i: (0, 0)),
        ],
        out_shape=[
            jax.ShapeDtypeStruct((ROWS, O), f32),
            jax.ShapeDtypeStruct((ROWS, O), f32),
            jax.ShapeDtypeStruct((8, O), f32),
        ],
    )(y1, stats1, fuse_g[None, :], fuse_beta[None, :], w1, b1[None, :])

    y3, stats3 = pl.pallas_call(
        functools.partial(_k3_body, count=count),
        grid=(RBLK,),
        in_specs=row_specs,
        out_specs=[
            pl.BlockSpec((NT, O), lambda i: (i, 0)),
            pl.BlockSpec((8, O), lambda i: (0, 0)),
        ],
        out_shape=[
            jax.ShapeDtypeStruct((ROWS, O), f32),
            jax.ShapeDtypeStruct((8, O), f32),
        ],
    )(y2, stats2, g1[None, :], be1[None, :], w2, b2[None, :])

    out = pl.pallas_call(
        functools.partial(_k4_body, count=count),
        grid=(B, NBLK),
        in_specs=[
            pl.BlockSpec((NT, O), lambda b, i: (b * NBLK + i, 0)),
            pl.BlockSpec((NT, O), lambda b, i: (b * NBLK + i, 0)),
            pl.BlockSpec((8, O), lambda b, i: (0, 0)),
            pl.BlockSpec((1, O), lambda b, i: (0, 0)),
            pl.BlockSpec((1, O), lambda b, i: (0, 0)),
        ],
        out_specs=pl.BlockSpec((1, O, NT), lambda b, i: (b, 0, i)),
        out_shape=jax.ShapeDtypeStruct((B, O, N), f32),
    )(y3, z, stats3, g2[None, :], be2[None, :])
    return out
